# Initial kernel scaffold; baseline (speedup 1.0000x reference)
import jax
import jax.numpy as jnp
from jax.experimental import pallas as pl


def kernel(cards, rank_emb, suit_emb):
    raise NotImplementedError("write your pallas kernel here")


# speedup vs baseline: 1.9697x; 1.9697x over previous
import functools

import jax
import jax.numpy as jnp
from jax import lax
from jax.experimental import pallas as pl
from jax.experimental.pallas import tpu as pltpu
from jax.experimental.pallas import tpu_sc as plsc

B, L, D = 4096, 50, 128
N = B * L
NRANK, NSUIT = 13, 4
NCOMBO = NRANK * NSUIT
NC, NS = 2, 16
NW = NC * NS
ROWS_PER_W = N // NW
CHUNK = 128
NCHUNK = ROWS_PER_W // CHUNK


def _combo_body(rank_ref, suit_ref, out_ref):
    rr = lax.broadcasted_iota(jnp.int32, (NCOMBO, NRANK), 0) // NSUIT
    rc = lax.broadcasted_iota(jnp.int32, (NCOMBO, NRANK), 1)
    oh_r = (rr == rc).astype(jnp.float32)
    sr = lax.broadcasted_iota(jnp.int32, (NCOMBO, NSUIT), 0) % NSUIT
    sc = lax.broadcasted_iota(jnp.int32, (NCOMBO, NSUIT), 1)
    oh_s = (sr == sc).astype(jnp.float32)
    out_ref[...] = (
        jnp.dot(oh_r, rank_ref[...], preferred_element_type=jnp.float32)
        + jnp.dot(oh_s, suit_ref[...], preferred_element_type=jnp.float32)
    )


def _make_combo(rank_emb, suit_emb):
    return pl.pallas_call(
        _combo_body,
        out_shape=jax.ShapeDtypeStruct((NCOMBO, D), jnp.float32),
    )(rank_emb, suit_emb)


_SC_MESH = plsc.VectorSubcoreMesh(core_axis_name="c", subcore_axis_name="s")


@functools.partial(
    pl.kernel,
    mesh=_SC_MESH,
    out_type=jax.ShapeDtypeStruct((N, D), jnp.float32),
    scratch_types=[
        pltpu.VMEM((CHUNK,), jnp.int32),
        pltpu.VMEM((CHUNK,), jnp.int32),
        pltpu.VMEM((CHUNK,), jnp.int32),
        pltpu.VMEM((CHUNK, D), jnp.float32),
        pltpu.SemaphoreType.DMA,
    ],
)
def _sc_lookup(r_hbm, s_hbm, combo_hbm, out_hbm, r_v, s_v, idx_v, rows_v, sem):
    wid = lax.axis_index("s") * NC + lax.axis_index("c")
    base = wid * ROWS_PER_W

    def chunk(g, carry):
        off = base + g * CHUNK
        pltpu.sync_copy(r_hbm.at[pl.ds(off, CHUNK)], r_v)
        pltpu.sync_copy(s_hbm.at[pl.ds(off, CHUNK)], s_v)
        for i in range(CHUNK // 16):
            sl = pl.ds(i * 16, 16)
            idx_v[sl] = r_v[sl] * NSUIT + s_v[sl]
        pltpu.async_copy(combo_hbm.at[idx_v], rows_v, sem).wait()
        pltpu.sync_copy(rows_v, out_hbm.at[pl.ds(off, CHUNK)])
        return carry

    lax.fori_loop(0, NCHUNK, chunk, 0)


def kernel(cards, rank_emb, suit_emb):
    combo = _make_combo(rank_emb, suit_emb)
    cf = cards.reshape(N, 2)
    out = _sc_lookup(cf[:, 0], cf[:, 1], combo)
    return out.reshape(B, L, D)


# speedup vs baseline: 2.0144x; 1.0227x over previous
import functools

import jax
import jax.numpy as jnp
from jax import lax
from jax.experimental import pallas as pl
from jax.experimental.pallas import tpu as pltpu
from jax.experimental.pallas import tpu_sc as plsc

B, L, D = 4096, 50, 128
N = B * L
NRANK, NSUIT = 13, 4
NCOMBO = NRANK * NSUIT
NC, NS = 2, 16
NW = NC * NS
ROWS_PER_W = N // NW
CHUNK = 128
NCHUNK = ROWS_PER_W // CHUNK
NBUF = 4
NTILE = NCHUNK // NBUF
NTAIL = NCHUNK - NTILE * NBUF


def _combo_body(rank_ref, suit_ref, out_ref):
    rr = lax.broadcasted_iota(jnp.int32, (NCOMBO, NRANK), 0) // NSUIT
    rc = lax.broadcasted_iota(jnp.int32, (NCOMBO, NRANK), 1)
    oh_r = (rr == rc).astype(jnp.float32)
    sr = lax.broadcasted_iota(jnp.int32, (NCOMBO, NSUIT), 0) % NSUIT
    sc = lax.broadcasted_iota(jnp.int32, (NCOMBO, NSUIT), 1)
    oh_s = (sr == sc).astype(jnp.float32)
    out_ref[...] = (
        jnp.dot(oh_r, rank_ref[...], preferred_element_type=jnp.float32)
        + jnp.dot(oh_s, suit_ref[...], preferred_element_type=jnp.float32)
    )


def _make_combo(rank_emb, suit_emb):
    return pl.pallas_call(
        _combo_body,
        out_shape=jax.ShapeDtypeStruct((NCOMBO, D), jnp.float32),
    )(rank_emb, suit_emb)


_SC_MESH = plsc.VectorSubcoreMesh(core_axis_name="c", subcore_axis_name="s")


@functools.partial(
    pl.kernel,
    mesh=_SC_MESH,
    out_type=jax.ShapeDtypeStruct((N, D), jnp.float32),
    scratch_types=[
        pltpu.VMEM((ROWS_PER_W,), jnp.int32),
        pltpu.VMEM((ROWS_PER_W,), jnp.int32),
        pltpu.VMEM((ROWS_PER_W,), jnp.int32),
        *[pltpu.VMEM((CHUNK, D), jnp.float32) for _ in range(NBUF)],
        *[pltpu.SemaphoreType.DMA for _ in range(2 * NBUF)],
    ],
)
def _sc_lookup(r_hbm, s_hbm, combo_hbm, out_hbm, r_v, s_v, idx_v, *bufs):
    rows = bufs[:NBUF]
    gsem = bufs[NBUF:2 * NBUF]
    ssem = bufs[2 * NBUF:]
    wid = lax.axis_index("s") * NC + lax.axis_index("c")
    rbase = pl.multiple_of(wid * ROWS_PER_W, ROWS_PER_W)

    pltpu.sync_copy(r_hbm.at[pl.ds(rbase, ROWS_PER_W)], r_v)
    pltpu.sync_copy(s_hbm.at[pl.ds(rbase, ROWS_PER_W)], s_v)

    def idx_body(g, carry):
        goff = pl.multiple_of(g * CHUNK, CHUNK)
        for i in range(CHUNK // 16):
            sl = pl.ds(goff + i * 16, 16)
            idx_v[sl] = r_v[sl] * NSUIT + s_v[sl]
        return carry

    lax.fori_loop(0, NCHUNK, idx_body, 0)

    def out_ref(g):
        return out_hbm.at[pl.ds(pl.multiple_of(rbase + g * CHUNK, CHUNK), CHUNK)]

    def idx_ref(g):
        return idx_v.at[pl.ds(pl.multiple_of(g * CHUNK, CHUNK), CHUNK)]

    def ring_body(t, carry):
        base = t * NBUF
        for b in range(NBUF):
            @pl.when(t > 0)
            def _():
                pltpu.make_async_copy(rows[b], out_ref(base + b), ssem[b]).wait()
            pltpu.make_async_copy(combo_hbm.at[idx_ref(base + b)],
                                  rows[b], gsem[b]).start()
        for b in range(NBUF):
            pltpu.make_async_copy(combo_hbm.at[idx_ref(base + b)],
                                  rows[b], gsem[b]).wait()
            pltpu.make_async_copy(rows[b], out_ref(base + b), ssem[b]).start()
        return carry

    lax.fori_loop(0, NTILE, ring_body, 0)

    tbase = NTILE * NBUF
    for b in range(NTAIL):
        pltpu.make_async_copy(rows[b], out_ref(tbase + b), ssem[b]).wait()
        pltpu.make_async_copy(combo_hbm.at[idx_ref(tbase + b)],
                              rows[b], gsem[b]).start()
    for b in range(NTAIL):
        pltpu.make_async_copy(combo_hbm.at[idx_ref(tbase + b)],
                              rows[b], gsem[b]).wait()
        pltpu.make_async_copy(rows[b], out_ref(tbase + b), ssem[b]).start()
    for b in range(NBUF):
        g = tbase + b if b < NTAIL else tbase - NBUF + b
        pltpu.make_async_copy(rows[b], out_ref(g), ssem[b]).wait()


def kernel(cards, rank_emb, suit_emb):
    combo = _make_combo(rank_emb, suit_emb)
    cf = cards.reshape(N, 2)
    out = _sc_lookup(cf[:, 0], cf[:, 1], combo)
    return out.reshape(B, L, D)


# speedup vs baseline: 6.5300x; 3.2417x over previous
import functools

import jax
import jax.numpy as jnp
from jax import lax
from jax.experimental import pallas as pl
from jax.experimental.pallas import tpu as pltpu
from jax.experimental.pallas import tpu_sc as plsc

B, L, D = 4096, 50, 128
N = B * L
NRANK, NSUIT = 13, 4
NCOMBO = NRANK * NSUIT
NC, NS = 2, 16
NW = NC * NS
ROWS_PER_W = N // NW
CHUNK = 128
NCHUNK = ROWS_PER_W // CHUNK
NBUF = 4
NTILE = NCHUNK // NBUF
NTAIL = NCHUNK - NTILE * NBUF


def _combo_body(rank_ref, suit_ref, out_ref):
    rr = lax.broadcasted_iota(jnp.int32, (NCOMBO, NRANK), 0) // NSUIT
    rc = lax.broadcasted_iota(jnp.int32, (NCOMBO, NRANK), 1)
    oh_r = (rr == rc).astype(jnp.float32)
    sr = lax.broadcasted_iota(jnp.int32, (NCOMBO, NSUIT), 0) % NSUIT
    sc = lax.broadcasted_iota(jnp.int32, (NCOMBO, NSUIT), 1)
    oh_s = (sr == sc).astype(jnp.float32)
    out_ref[...] = (
        jnp.dot(oh_r, rank_ref[...], preferred_element_type=jnp.float32)
        + jnp.dot(oh_s, suit_ref[...], preferred_element_type=jnp.float32)
    )


def _make_combo(rank_emb, suit_emb):
    return pl.pallas_call(
        _combo_body,
        out_shape=jax.ShapeDtypeStruct((NCOMBO, D), jnp.float32),
    )(rank_emb, suit_emb)


_SC_MESH = plsc.VectorSubcoreMesh(core_axis_name="c", subcore_axis_name="s")


@functools.partial(
    pl.kernel,
    mesh=_SC_MESH,
    out_type=jax.ShapeDtypeStruct((N, D), jnp.float32),
    scratch_types=[
        pltpu.VMEM((ROWS_PER_W,), jnp.int32),
        pltpu.VMEM((ROWS_PER_W,), jnp.int32),
        pltpu.VMEM((ROWS_PER_W,), jnp.int32),
        *[pltpu.VMEM((CHUNK, D), jnp.float32) for _ in range(NBUF)],
        pltpu.VMEM_SHARED((NCOMBO, D), jnp.float32),
        *[pltpu.SemaphoreType.DMA for _ in range(2 * NBUF)],
    ],
)
def _sc_lookup(r_hbm, s_hbm, combo_hbm, out_hbm, r_v, s_v, idx_v, *bufs):
    rows = bufs[:NBUF]
    combo_sp = bufs[NBUF]
    gsem = bufs[NBUF + 1:2 * NBUF + 1]
    ssem = bufs[2 * NBUF + 1:]
    sid = lax.axis_index("s")
    wid = sid * NC + lax.axis_index("c")
    rbase = pl.multiple_of(wid * ROWS_PER_W, ROWS_PER_W)

    @pl.when(sid == 0)
    def _():
        pltpu.sync_copy(combo_hbm, combo_sp)
    plsc.subcore_barrier()

    pltpu.sync_copy(r_hbm.at[pl.ds(rbase, ROWS_PER_W)], r_v)
    pltpu.sync_copy(s_hbm.at[pl.ds(rbase, ROWS_PER_W)], s_v)

    def idx_body(g, carry):
        goff = pl.multiple_of(g * CHUNK, CHUNK)
        for i in range(CHUNK // 16):
            sl = pl.ds(goff + i * 16, 16)
            idx_v[sl] = r_v[sl] * NSUIT + s_v[sl]
        return carry

    lax.fori_loop(0, NCHUNK, idx_body, 0)

    def out_ref(g):
        return out_hbm.at[pl.ds(pl.multiple_of(rbase + g * CHUNK, CHUNK), CHUNK)]

    def idx_ref(g):
        return idx_v.at[pl.ds(pl.multiple_of(g * CHUNK, CHUNK), CHUNK)]

    def ring_body(t, carry):
        base = t * NBUF
        for b in range(NBUF):
            @pl.when(t > 0)
            def _():
                pltpu.make_async_copy(rows[b], out_ref(base + b), ssem[b]).wait()
            pltpu.make_async_copy(combo_sp.at[idx_ref(base + b)],
                                  rows[b], gsem[b]).start()
        for b in range(NBUF):
            pltpu.make_async_copy(combo_sp.at[idx_ref(base + b)],
                                  rows[b], gsem[b]).wait()
            pltpu.make_async_copy(rows[b], out_ref(base + b), ssem[b]).start()
        return carry

    lax.fori_loop(0, NTILE, ring_body, 0)

    tbase = NTILE * NBUF
    for b in range(NTAIL):
        pltpu.make_async_copy(rows[b], out_ref(tbase + b), ssem[b]).wait()
        pltpu.make_async_copy(combo_sp.at[idx_ref(tbase + b)],
                              rows[b], gsem[b]).start()
    for b in range(NTAIL):
        pltpu.make_async_copy(combo_sp.at[idx_ref(tbase + b)],
                              rows[b], gsem[b]).wait()
        pltpu.make_async_copy(rows[b], out_ref(tbase + b), ssem[b]).start()
    for b in range(NBUF):
        g = tbase + b if b < NTAIL else tbase - NBUF + b
        pltpu.make_async_copy(rows[b], out_ref(g), ssem[b]).wait()


def kernel(cards, rank_emb, suit_emb):
    combo = _make_combo(rank_emb, suit_emb)
    cf = cards.reshape(N, 2)
    out = _sc_lookup(cf[:, 0], cf[:, 1], combo)
    return out.reshape(B, L, D)
